# baseline (speedup 1.0000x reference)
import functools

import jax
import jax.numpy as jnp
from jax import lax
from jax.experimental import pallas as pl
from jax.experimental.pallas import tpu as pltpu
from jax.experimental.pallas import tpu_sc as plsc

_LANES = 16
_SUB = 32


def _emb_lookup(idx, word_emb, pos_emb, num_cores, num_subcores):
    B, T = idx.shape
    V, D = word_emb.shape
    NW = num_cores * num_subcores
    CH = T // NW
    NSUB = CH // _SUB
    NU = B * NSUB

    mesh = plsc.VectorSubcoreMesh(core_axis_name="c", subcore_axis_name="s")

    @functools.partial(
        pl.kernel,
        mesh=mesh,
        out_type=jax.ShapeDtypeStruct((B, T, D), jnp.float32),
        scratch_types=[
            pltpu.VMEM((B, CH), jnp.int32),
            pltpu.VMEM((CH, D), jnp.float32),
            pltpu.VMEM((_SUB, D), jnp.float32),
            pltpu.VMEM((_SUB, D), jnp.float32),
            pltpu.SemaphoreType.DMA,
            pltpu.SemaphoreType.DMA,
            pltpu.SemaphoreType.DMA,
            pltpu.SemaphoreType.DMA,
        ],
    )
    def emb_kernel(idx_hbm, word_hbm, pos_hbm, out_hbm,
                   idx_v, pos_v, rows0, rows1, g0, g1, o0, o1):
        wid = lax.axis_index("s") * num_cores + lax.axis_index("c")
        t0 = wid * CH
        pltpu.sync_copy(pos_hbm.at[pl.ds(t0, CH)], pos_v)
        for b in range(B):
            pltpu.sync_copy(idx_hbm.at[b, pl.ds(t0, CH)], idx_v.at[b])

        rows = [rows0, rows1]
        gsem = [g0, g1]
        osem = [o0, o1]
        gathers = [None] * NU
        outs = [None] * NU

        def start_gather(u):
            b, s = divmod(u, NSUB)
            gathers[u] = pltpu.async_copy(
                word_hbm.at[idx_v.at[b, pl.ds(s * _SUB, _SUB)]],
                rows[u % 2], gsem[u % 2])

        start_gather(0)
        for u in range(NU):
            if u + 1 < NU:
                if u + 1 >= 2:
                    outs[u - 1].wait()
                start_gather(u + 1)
            gathers[u].wait()
            b, s = divmod(u, NSUB)
            base = s * _SUB
            buf = rows[u % 2]

            def add_row(i, carry, _buf=buf, _base=base):
                for k in range(D // _LANES):
                    sl = pl.ds(k * _LANES, _LANES)
                    plsc.addupdate(_buf.at[i, sl], pos_v[_base + i, sl])
                return carry

            lax.fori_loop(0, _SUB, add_row, 0)
            outs[u] = pltpu.async_copy(
                buf, out_hbm.at[b, pl.ds(t0 + base, _SUB)], osem[u % 2])
        outs[NU - 2].wait()
        outs[NU - 1].wait()

    return emb_kernel(idx, word_emb, pos_emb)


def kernel(idx, word_emb, pos_emb):
    idx = jnp.asarray(idx, jnp.int32)
    return _emb_lookup(idx, word_emb, pos_emb, num_cores=2, num_subcores=16)


# speedup vs baseline: 1.1525x; 1.1525x over previous
import functools

import jax
import jax.numpy as jnp
from jax import lax
from jax.experimental import pallas as pl
from jax.experimental.pallas import tpu as pltpu
from jax.experimental.pallas import tpu_sc as plsc

_LANES = 16
_SUB = 32


def _emb_lookup(idx, word_emb, pos_emb, num_cores, num_subcores):
    B, T = idx.shape
    V, D = word_emb.shape
    NW = num_cores * num_subcores
    CH = T // NW
    NSUB = CH // _SUB
    NU = B * NSUB

    mesh = plsc.VectorSubcoreMesh(core_axis_name="c", subcore_axis_name="s")

    @functools.partial(
        pl.kernel,
        mesh=mesh,
        out_type=jax.ShapeDtypeStruct((B, T, D), jnp.float32),
        scratch_types=[
            pltpu.VMEM((B, CH), jnp.int32),
            pltpu.VMEM((CH, D), jnp.float32),
            pltpu.VMEM((_SUB, D), jnp.float32),
            pltpu.VMEM((_SUB, D), jnp.float32),
            pltpu.SemaphoreType.DMA,
            pltpu.SemaphoreType.DMA,
            pltpu.SemaphoreType.DMA,
            pltpu.SemaphoreType.DMA,
        ],
    )
    def emb_kernel(idx_hbm, word_hbm, pos_hbm, out_hbm,
                   idx_v, pos_v, rows0, rows1, g0, g1, o0, o1):
        wid = lax.axis_index("s") * num_cores + lax.axis_index("c")
        t0 = wid * CH
        pltpu.sync_copy(pos_hbm.at[pl.ds(t0, CH)], pos_v)
        for b in range(B):
            pltpu.sync_copy(idx_hbm.at[b, pl.ds(t0, CH)], idx_v.at[b])

        rows = [rows0, rows1]
        gsem = [g0, g1]
        osem = [o0, o1]
        gathers = [None] * NU
        outs = [None] * NU

        def start_gather(u):
            b, s = divmod(u, NSUB)
            gathers[u] = pltpu.async_copy(
                word_hbm.at[idx_v.at[b, pl.ds(s * _SUB, _SUB)]],
                rows[u % 2], gsem[u % 2])

        start_gather(0)
        for u in range(NU):
            if u + 1 < NU:
                if u + 1 >= 2:
                    outs[u - 1].wait()
                start_gather(u + 1)
            gathers[u].wait()
            b, s = divmod(u, NSUB)
            base = s * _SUB
            buf = rows[u % 2]

            @plsc.parallel_loop(0, _SUB, unroll=2)
            def add_row(i, _buf=buf, _base=base):
                for k in range(D // _LANES):
                    sl = pl.ds(k * _LANES, _LANES)
                    plsc.addupdate(_buf.at[i, sl], pos_v[_base + i, sl])

            outs[u] = pltpu.async_copy(
                buf, out_hbm.at[b, pl.ds(t0 + base, _SUB)], osem[u % 2])
        outs[NU - 2].wait()
        outs[NU - 1].wait()

    return emb_kernel(idx, word_emb, pos_emb)


def kernel(idx, word_emb, pos_emb):
    idx = jnp.asarray(idx, jnp.int32)
    return _emb_lookup(idx, word_emb, pos_emb, num_cores=2, num_subcores=16)


# speedup vs baseline: 1.2343x; 1.0710x over previous
import functools

import jax
import jax.numpy as jnp
from jax import lax
from jax.experimental import pallas as pl
from jax.experimental.pallas import tpu as pltpu
from jax.experimental.pallas import tpu_sc as plsc

_LANES = 16
_SUB = 32
_NBUF = 3


def _emb_lookup(idx, word_emb, pos_emb, num_cores, num_subcores):
    B, T = idx.shape
    V, D = word_emb.shape
    NW = num_cores * num_subcores
    CH = T // NW
    NSUB = CH // _SUB
    NU = B * NSUB

    mesh = plsc.VectorSubcoreMesh(core_axis_name="c", subcore_axis_name="s")

    @functools.partial(
        pl.kernel,
        mesh=mesh,
        out_type=jax.ShapeDtypeStruct((B, T, D), jnp.float32),
        scratch_types=[
            pltpu.VMEM((B, CH), jnp.int32),
            pltpu.VMEM((CH, D), jnp.float32),
        ] + [pltpu.VMEM((_SUB, D), jnp.float32)] * _NBUF
          + [pltpu.SemaphoreType.DMA] * (2 * _NBUF),
    )
    def emb_kernel(idx_hbm, word_hbm, pos_hbm, out_hbm, idx_v, pos_v,
                   *bufs_and_sems):
        rows = list(bufs_and_sems[:_NBUF])
        gsem = list(bufs_and_sems[_NBUF:2 * _NBUF])
        osem = list(bufs_and_sems[2 * _NBUF:])

        wid = lax.axis_index("s") * num_cores + lax.axis_index("c")
        t0 = wid * CH
        for b in range(B):
            pltpu.sync_copy(idx_hbm.at[b, pl.ds(t0, CH)], idx_v.at[b])

        gathers = [None] * NU
        outs = [None] * NU

        def gath(u):
            b, s = divmod(u, NSUB)
            gathers[u] = pltpu.async_copy(
                word_hbm.at[idx_v.at[b, pl.ds(s * _SUB, _SUB)]],
                rows[u % _NBUF], gsem[u % _NBUF])

        def outw(u):
            b, s = divmod(u, NSUB)
            outs[u] = pltpu.async_copy(
                rows[u % _NBUF], out_hbm.at[b, pl.ds(t0 + s * _SUB, _SUB)],
                osem[u % _NBUF])

        gath(0)
        gath(1)
        pltpu.sync_copy(pos_hbm.at[pl.ds(t0, CH)], pos_v)
        for u in range(NU):
            gathers[u].wait()
            buf = rows[u % _NBUF]
            base = (u % NSUB) * _SUB

            @plsc.parallel_loop(0, _SUB, unroll=2)
            def add_row(i, _buf=buf, _base=base):
                for k in range(D // _LANES):
                    sl = pl.ds(k * _LANES, _LANES)
                    plsc.addupdate(_buf.at[i, sl], pos_v[_base + i, sl])

            outw(u)
            if u + 2 < NU:
                if u >= 1:
                    outs[u - 1].wait()
                gath(u + 2)
        for u in range(max(0, NU - 2), NU):
            outs[u].wait()

    return emb_kernel(idx, word_emb, pos_emb)


def kernel(idx, word_emb, pos_emb):
    idx = jnp.asarray(idx, jnp.int32)
    return _emb_lookup(idx, word_emb, pos_emb, num_cores=2, num_subcores=16)


# speedup vs baseline: 1.2764x; 1.0340x over previous
import functools

import jax
import jax.numpy as jnp
from jax import lax
from jax.experimental import pallas as pl
from jax.experimental.pallas import tpu as pltpu
from jax.experimental.pallas import tpu_sc as plsc

_LANES = 16
_SUB = 32
_NBUF = 3


def _emb_lookup(idx, word_emb, pos_emb, num_cores, num_subcores):
    B, T = idx.shape
    V, D = word_emb.shape
    NW = num_cores * num_subcores
    CH = T // NW
    NSUB = CH // _SUB
    NU = B * NSUB

    mesh = plsc.VectorSubcoreMesh(core_axis_name="c", subcore_axis_name="s")

    @functools.partial(
        pl.kernel,
        mesh=mesh,
        out_type=jax.ShapeDtypeStruct((B, T, D), jnp.float32),
        scratch_types=[
            pltpu.VMEM((B, CH), jnp.int32),
            pltpu.VMEM((CH, D), jnp.float32),
        ] + [pltpu.VMEM((_SUB, D), jnp.float32)] * _NBUF
          + [pltpu.SemaphoreType.DMA] * (2 * _NBUF + 2),
    )
    def emb_kernel(idx_hbm, word_hbm, pos_hbm, out_hbm, idx_v, pos_v,
                   *bufs_and_sems):
        rows = list(bufs_and_sems[:_NBUF])
        gsem = list(bufs_and_sems[_NBUF:2 * _NBUF])
        osem = list(bufs_and_sems[2 * _NBUF:3 * _NBUF])
        isem = bufs_and_sems[3 * _NBUF]
        psem = bufs_and_sems[3 * _NBUF + 1]

        wid = lax.axis_index("s") * num_cores + lax.axis_index("c")
        t0 = wid * CH
        icopies = [
            pltpu.async_copy(idx_hbm.at[b, pl.ds(t0, CH)], idx_v.at[b], isem)
            for b in range(B)
        ]

        gathers = [None] * NU
        outs = [None] * NU

        def gath(u):
            b, s = divmod(u, NSUB)
            gathers[u] = pltpu.async_copy(
                word_hbm.at[idx_v.at[b, pl.ds(s * _SUB, _SUB)]],
                rows[u % _NBUF], gsem[u % _NBUF])

        def outw(u):
            b, s = divmod(u, NSUB)
            outs[u] = pltpu.async_copy(
                rows[u % _NBUF], out_hbm.at[b, pl.ds(t0 + s * _SUB, _SUB)],
                osem[u % _NBUF])

        pcopy = pltpu.async_copy(pos_hbm.at[pl.ds(t0, CH)], pos_v, psem)
        for c in icopies:
            c.wait()
        gath(0)
        gath(1)
        pcopy.wait()
        for u in range(NU):
            gathers[u].wait()
            buf = rows[u % _NBUF]
            base = (u % NSUB) * _SUB

            @plsc.parallel_loop(0, _SUB, unroll=2)
            def add_row(i, _buf=buf, _base=base):
                for k in range(D // _LANES):
                    sl = pl.ds(k * _LANES, _LANES)
                    plsc.addupdate(_buf.at[i, sl], pos_v[_base + i, sl])

            outw(u)
            if u + 2 < NU:
                if u >= 1:
                    outs[u - 1].wait()
                gath(u + 2)
        for u in range(max(0, NU - 2), NU):
            outs[u].wait()

    return emb_kernel(idx, word_emb, pos_emb)


def kernel(idx, word_emb, pos_emb):
    idx = jnp.asarray(idx, jnp.int32)
    return _emb_lookup(idx, word_emb, pos_emb, num_cores=2, num_subcores=16)


# speedup vs baseline: 1.3151x; 1.0303x over previous
import functools

import jax
import jax.numpy as jnp
from jax import lax
from jax.experimental import pallas as pl
from jax.experimental.pallas import tpu as pltpu
from jax.experimental.pallas import tpu_sc as plsc

_LANES = 16
_SUB = 8
_NSET = 3


def _emb_lookup(idx, word_emb, pos_emb, num_cores, num_subcores):
    B, T = idx.shape
    V, D = word_emb.shape
    NW = num_cores * num_subcores
    CH = T // NW
    NSU = CH // _SUB

    mesh = plsc.VectorSubcoreMesh(core_axis_name="c", subcore_axis_name="s")

    @functools.partial(
        pl.kernel,
        mesh=mesh,
        out_type=jax.ShapeDtypeStruct((B, T, D), jnp.float32),
        scratch_types=[
            pltpu.VMEM((B, CH), jnp.int32),
            pltpu.VMEM((CH, D), jnp.float32),
        ] + [pltpu.VMEM((_SUB, D), jnp.float32)] * (B * _NSET)
          + [pltpu.SemaphoreType.DMA] * (2 * _NSET + 2),
    )
    def emb_kernel(idx_hbm, word_hbm, pos_hbm, out_hbm, idx_v, pos_v,
                   *bufs_and_sems):
        sets = [list(bufs_and_sems[m * B:(m + 1) * B]) for m in range(_NSET)]
        rest = bufs_and_sems[B * _NSET:]
        gsem = list(rest[:_NSET])
        osem = list(rest[_NSET:2 * _NSET])
        isem = rest[2 * _NSET]
        psem = rest[2 * _NSET + 1]

        wid = lax.axis_index("s") * num_cores + lax.axis_index("c")
        t0 = wid * CH
        icopies = [
            pltpu.async_copy(idx_hbm.at[b, pl.ds(t0, CH)], idx_v.at[b], isem)
            for b in range(B)
        ]
        pcopy = pltpu.async_copy(pos_hbm.at[pl.ds(t0, CH)], pos_v, psem)

        gathers = [None] * NSU
        outs = [None] * NSU

        def gath(su):
            m = su % _NSET
            gathers[su] = [
                pltpu.async_copy(
                    word_hbm.at[idx_v.at[b, pl.ds(su * _SUB, _SUB)]],
                    sets[m][b], gsem[m])
                for b in range(B)
            ]

        def outw(su):
            m = su % _NSET
            outs[su] = [
                pltpu.async_copy(
                    sets[m][b], out_hbm.at[b, pl.ds(t0 + su * _SUB, _SUB)],
                    osem[m])
                for b in range(B)
            ]

        for c in icopies:
            c.wait()
        gath(0)
        gath(1)
        pcopy.wait()
        for su in range(NSU):
            for c in gathers[su]:
                c.wait()
            m = su % _NSET
            base = su * _SUB
            bufs = sets[m]

            @plsc.parallel_loop(0, _SUB, unroll=1)
            def add_row(i, _bufs=bufs, _base=base):
                for k in range(D // _LANES):
                    sl = pl.ds(k * _LANES, _LANES)
                    p = pos_v[_base + i, sl]
                    for b in range(B):
                        plsc.addupdate(_bufs[b].at[i, sl], p)

            outw(su)
            if su + 2 < NSU:
                if su >= 1:
                    for c in outs[su - 1]:
                        c.wait()
                gath(su + 2)
        for su in range(max(0, NSU - 2), NSU):
            for c in outs[su]:
                c.wait()

    return emb_kernel(idx, word_emb, pos_emb)


def kernel(idx, word_emb, pos_emb):
    idx = jnp.asarray(idx, jnp.int32)
    return _emb_lookup(idx, word_emb, pos_emb, num_cores=2, num_subcores=16)


# speedup vs baseline: 1.3206x; 1.0042x over previous
import functools

import jax
import jax.numpy as jnp
from jax import lax
from jax.experimental import pallas as pl
from jax.experimental.pallas import tpu as pltpu
from jax.experimental.pallas import tpu_sc as plsc

_LANES = 16
_SUB = 8
_NSET = 4
_PD = 3


def _emb_lookup(idx, word_emb, pos_emb, num_cores, num_subcores):
    B, T = idx.shape
    V, D = word_emb.shape
    NW = num_cores * num_subcores
    CH = T // NW
    NSU = CH // _SUB

    mesh = plsc.VectorSubcoreMesh(core_axis_name="c", subcore_axis_name="s")

    @functools.partial(
        pl.kernel,
        mesh=mesh,
        out_type=jax.ShapeDtypeStruct((B, T, D), jnp.float32),
        scratch_types=[
            pltpu.VMEM((B, CH), jnp.int32),
        ] + [pltpu.VMEM((_SUB, D), jnp.float32)] * ((B + 1) * _NSET)
          + [pltpu.SemaphoreType.DMA] * (3 * _NSET + 1),
    )
    def emb_kernel(idx_hbm, word_hbm, pos_hbm, out_hbm, idx_v, *bufs_and_sems):
        nb = B + 1
        sets = [list(bufs_and_sems[m * nb:(m + 1) * nb]) for m in range(_NSET)]
        rest = bufs_and_sems[nb * _NSET:]
        gsem = list(rest[:_NSET])
        osem = list(rest[_NSET:2 * _NSET])
        psem = list(rest[2 * _NSET:3 * _NSET])
        isem = rest[3 * _NSET]

        wid = lax.axis_index("s") * num_cores + lax.axis_index("c")
        t0 = wid * CH
        icopies = [
            pltpu.async_copy(idx_hbm.at[b, pl.ds(t0, CH)], idx_v.at[b], isem)
            for b in range(B)
        ]

        fills = [None] * NSU
        outs = [None] * NSU

        def fill(su):
            m = su % _NSET
            pcopy = pltpu.async_copy(
                pos_hbm.at[pl.ds(t0 + su * _SUB, _SUB)], sets[m][B], psem[m])
            gcopies = [
                pltpu.async_copy(
                    word_hbm.at[idx_v.at[b, pl.ds(su * _SUB, _SUB)]],
                    sets[m][b], gsem[m])
                for b in range(B)
            ]
            fills[su] = gcopies + [pcopy]

        def outw(su):
            m = su % _NSET
            outs[su] = [
                pltpu.async_copy(
                    sets[m][b], out_hbm.at[b, pl.ds(t0 + su * _SUB, _SUB)],
                    osem[m])
                for b in range(B)
            ]

        for c in icopies:
            c.wait()
        for su in range(_PD):
            fill(su)
        for su in range(NSU):
            for c in fills[su]:
                c.wait()
            m = su % _NSET
            bufs = sets[m]

            @plsc.parallel_loop(0, _SUB, unroll=1)
            def add_row(i, _bufs=bufs):
                for k in range(D // _LANES):
                    sl = pl.ds(k * _LANES, _LANES)
                    p = _bufs[B][i, sl]
                    for b in range(B):
                        plsc.addupdate(_bufs[b].at[i, sl], p)

            outw(su)
            if su + _PD < NSU:
                back = su - (_NSET - _PD)
                if back >= 0:
                    for c in outs[back]:
                        c.wait()
                fill(su + _PD)
        for su in range(max(0, NSU - _NSET), NSU):
            for c in outs[su]:
                c.wait()

    return emb_kernel(idx, word_emb, pos_emb)


def kernel(idx, word_emb, pos_emb):
    idx = jnp.asarray(idx, jnp.int32)
    return _emb_lookup(idx, word_emb, pos_emb, num_cores=2, num_subcores=16)
